# Initial kernel scaffold; baseline (speedup 1.0000x reference)
import jax
import jax.numpy as jnp
from jax.experimental import pallas as pl


def kernel(w, table):
    raise NotImplementedError("write your pallas kernel here")


# speedup vs baseline: 10.9949x; 10.9949x over previous
import functools

import jax
import jax.numpy as jnp
from jax import lax
from jax.experimental import pallas as pl
from jax.experimental.pallas import tpu as pltpu
from jax.experimental.pallas import tpu_sc as plsc

BATCH = 16384
SEQ = 201
DIM = 64
BFLAT = BATCH * SEQ
NC, NS = 2, 16
NW = NC * NS
ROWS_PER_W = BFLAT // NW
CHUNK = 512
IW = 128
IPC = CHUNK // IW
CHUNKS_PER_W = ROWS_PER_W // CHUNK


def _make_gather():
    mesh = plsc.VectorSubcoreMesh(core_axis_name="c", subcore_axis_name="s")

    @functools.partial(
        pl.kernel,
        mesh=mesh,
        out_type=jax.ShapeDtypeStruct((BFLAT, DIM), jnp.float32),
        scratch_types=[
            pltpu.VMEM((IPC, IW), jnp.int32),
            pltpu.VMEM((CHUNK, DIM), jnp.float32),
            pltpu.SemaphoreType.DMA,
        ],
        compiler_params=pltpu.CompilerParams(use_tc_tiling_on_sc=False),
    )
    def gather_kernel(idx_hbm, table_hbm, out_hbm, idx_v, rows_v, sem):
        wid = lax.axis_index("s") * NC + lax.axis_index("c")

        def body(g, carry):
            c = wid * CHUNKS_PER_W + g
            pltpu.sync_copy(idx_hbm.at[pl.ds(c * IPC, IPC)], idx_v)
            copies = []
            for j in range(IPC):
                copies.append(
                    pltpu.async_copy(
                        table_hbm.at[idx_v.at[j]],
                        rows_v.at[pl.ds(j * IW, IW)],
                        sem,
                    )
                )
            for cp in copies:
                cp.wait()
            pltpu.sync_copy(rows_v, out_hbm.at[pl.ds(c * CHUNK, CHUNK)])
            return carry

        lax.fori_loop(0, CHUNKS_PER_W, body, 0)

    return gather_kernel


_gather = _make_gather()


@jax.jit
def kernel(w, table):
    idx = w.astype(jnp.int32).reshape(BFLAT // IW, IW)
    out = _gather(idx, table)
    return out.reshape(BATCH, SEQ, DIM)


# speedup vs baseline: 12.0114x; 1.0924x over previous
import functools

import jax
import jax.numpy as jnp
from jax import lax
from jax.experimental import pallas as pl
from jax.experimental.pallas import tpu as pltpu
from jax.experimental.pallas import tpu_sc as plsc

BATCH = 16384
SEQ = 201
DIM = 64
BFLAT = BATCH * SEQ
NC, NS = 2, 16
NW = NC * NS
ROWS_PER_W = BFLAT // NW
CHUNK = 512
IW = 128
IPC = CHUNK // IW
CHUNKS_PER_W = ROWS_PER_W // CHUNK
NB = 3
OUTER = CHUNKS_PER_W // NB


def _make_gather():
    mesh = plsc.VectorSubcoreMesh(core_axis_name="c", subcore_axis_name="s")

    @functools.partial(
        pl.kernel,
        mesh=mesh,
        out_type=jax.ShapeDtypeStruct((BFLAT, DIM), jnp.float32),
        scratch_types=[
            [pltpu.VMEM((IPC, IW), jnp.int32) for _ in range(NB)],
            [pltpu.VMEM((CHUNK, DIM), jnp.float32) for _ in range(NB)],
            [pltpu.SemaphoreType.DMA for _ in range(NB)],
            [pltpu.SemaphoreType.DMA for _ in range(NB)],
        ],
        compiler_params=pltpu.CompilerParams(use_tc_tiling_on_sc=False),
    )
    def gather_kernel(idx_hbm, table_hbm, out_hbm, idx_v, rows_v, gsem, ssem):
        wid = lax.axis_index("s") * NC + lax.axis_index("c")
        base = wid * CHUNKS_PER_W

        def fire_gathers(b, c):
            pltpu.sync_copy(idx_hbm.at[pl.ds((base + c) * IPC, IPC)], idx_v[b])
            for j in range(IPC):
                pltpu.async_copy(
                    table_hbm.at[idx_v[b].at[j]],
                    rows_v[b].at[pl.ds(j * IW, IW)],
                    gsem[b],
                )

        def wait_gathers(b):
            for j in range(IPC):
                pltpu.make_async_copy(
                    table_hbm.at[idx_v[b].at[j]],
                    rows_v[b].at[pl.ds(j * IW, IW)],
                    gsem[b],
                ).wait()

        def fire_store(b, c):
            pltpu.async_copy(
                rows_v[b], out_hbm.at[pl.ds((base + c) * CHUNK, CHUNK)], ssem[b]
            )

        def wait_store(b, c):
            pltpu.make_async_copy(
                rows_v[b], out_hbm.at[pl.ds((base + c) * CHUNK, CHUNK)], ssem[b]
            ).wait()

        for b in range(NB):
            fire_gathers(b, b)

        def body(g, carry):
            c0 = g * NB
            for b in range(NB):
                wait_gathers(b)
                fire_store(b, c0 + b)

            @pl.when(g < OUTER - 1)
            def _prefetch():
                for b in range(NB):
                    wait_store(b, c0 + b)
                    fire_gathers(b, c0 + NB + b)

            return carry

        lax.fori_loop(0, OUTER, body, 0)
        for b in range(NB):
            wait_store(b, (OUTER - 1) * NB + b)

    return gather_kernel


_gather = _make_gather()


@jax.jit
def kernel(w, table):
    idx = w.astype(jnp.int32).reshape(BFLAT // IW, IW)
    out = _gather(idx, table)
    return out.reshape(BATCH, SEQ, DIM)
